# baseline (speedup 1.0000x reference)
import functools

import jax
import jax.numpy as jnp
from jax import lax
from jax.experimental import pallas as pl
from jax.experimental.pallas import tpu as pltpu
from jax.experimental.pallas import tpu_sc as plsc

_NUM_EMB = 100000
_D = 128
_B = 16384
_BLK = 4096

_NC = 2
_NS = 16
_NW = _NC * _NS
_EPW = _NUM_EMB // _NW
_WTBL = 3136
_LROWS = 28
_CH = 128


def _dense_body(ctx_ref, emb_ref, wc_ref, bc_ref, wd_ref, bd_ref, out_ref):
    ctx = ctx_ref[...]
    emb = emb_ref[...]
    ct = lax.dot_general(ctx, wc_ref[...], (((1,), (1,)), ((), ())),
                         preferred_element_type=jnp.float32) + bc_ref[...]
    ct = 1.0 / (1.0 + jnp.exp(-ct))
    dl = lax.dot_general(emb, wd_ref[...], (((1,), (1,)), ((), ())),
                         preferred_element_type=jnp.float32) + bd_ref[...]
    dl = dl * ct
    dl = 1.0 / (1.0 + jnp.exp(-dl))
    u = dl * emb + (1.0 - dl) * ct
    nrm = jnp.sqrt(jnp.sum(u * u, axis=1, keepdims=True))
    out_ref[...] = u / jnp.maximum(nrm, 1e-12)


_dense = pl.pallas_call(
    _dense_body,
    grid=(_B // _BLK,),
    in_specs=[
        pl.BlockSpec((_BLK, _D), lambda i: (i, 0)),
        pl.BlockSpec((_BLK, _D), lambda i: (i, 0)),
        pl.BlockSpec((_D, _D), lambda i: (0, 0)),
        pl.BlockSpec((1, _D), lambda i: (0, 0)),
        pl.BlockSpec((_D, _D), lambda i: (0, 0)),
        pl.BlockSpec((1, _D), lambda i: (0, 0)),
    ],
    out_specs=pl.BlockSpec((_BLK, _D), lambda i: (i, 0)),
    out_shape=jax.ShapeDtypeStruct((_B, _D), jnp.float32),
)


@functools.partial(
    pl.kernel,
    mesh=plsc.VectorSubcoreMesh(core_axis_name="c", subcore_axis_name="s"),
    out_type=(
        jax.ShapeDtypeStruct((_NW, _LROWS, _CH), jnp.int32),
        jax.ShapeDtypeStruct((_NW, _LROWS, _CH), jnp.int32),
        jax.ShapeDtypeStruct((_NW, 128), jnp.int32),
    ),
    scratch_types=[
        pltpu.VMEM((_B,), jnp.int32),
        pltpu.VMEM((_WTBL,), jnp.int32),
        pltpu.VMEM((_LROWS, _CH), jnp.int32),
        pltpu.VMEM((_LROWS, _CH), jnp.int32),
        pltpu.VMEM((128,), jnp.int32),
    ],
    compiler_params=pltpu.CompilerParams(needs_layout_passes=False),
)
def _winner_k(idx_hbm, el_hbm, bl_hbm, cnt_hbm,
              idx_v, wtbl, el2, bl2, cnt_v):
    wid = lax.axis_index("s") * _NC + lax.axis_index("c")
    lo = wid * _EPW
    iot = lax.iota(jnp.int32, 16)

    pltpu.sync_copy(idx_hbm, idx_v)

    zero = jnp.zeros((16,), jnp.int32)

    def _clear(i, c):
        wtbl[pl.ds(i * 16, 16)] = zero
        return c
    lax.fori_loop(0, _WTBL // 16, _clear, 0)

    def _scan(c4, carry):
        for u in range(4):
            c = c4 * 4 + u
            iv = idx_v[pl.ds(c * 16, 16)]
            _, lastm = plsc.scan_count(iv)
            own = (iv >= lo) & (iv < lo + _EPW) & lastm
            loc = jnp.where(own, iv - lo, _EPW)
            bv = c * 16 + iot + 1
            plsc.store_scatter(wtbl, [loc], bv, mask=own)
        return carry
    lax.fori_loop(0, _B // 64, _scan, 0)

    def _compact(c, n):
        wv = wtbl[pl.ds(c * 16, 16)]
        m = wv > 0
        mi = m.astype(jnp.int32)
        ev = lo + c * 16 + iot
        incl = plsc.cumsum(mi)
        pos = n + incl - 1
        r = lax.shift_right_logical(pos, 7)
        col = pos & 127
        plsc.store_scatter(el2, [r, col], ev, mask=m)
        plsc.store_scatter(bl2, [r, col], wv - 1, mask=m)
        return n + lax.reduce_sum(mi, (0,))
    n = lax.fori_loop(0, _WTBL // 16, _compact, jnp.int32(0))

    last = jnp.maximum(n - 1, 0)
    lr = jnp.full((16,), lax.shift_right_logical(last, 7), jnp.int32)
    lc = jnp.full((16,), last & 127, jnp.int32)
    e0 = plsc.load_gather(el2, [lr, lc])
    b0 = plsc.load_gather(bl2, [lr, lc])
    for k in range(8):
        posv = n + k * 16 + iot
        pr = lax.shift_right_logical(posv, 7)
        pc = posv & 127
        plsc.store_scatter(el2, [pr, pc], e0)
        plsc.store_scatter(bl2, [pr, pc], b0)

    cnt_v[pl.ds(0, 16)] = jnp.full((16,), n, jnp.int32)
    pltpu.sync_copy(el2, el_hbm.at[wid])
    pltpu.sync_copy(bl2, bl_hbm.at[wid])
    pltpu.sync_copy(cnt_v, cnt_hbm.at[wid])


@functools.partial(
    pl.kernel,
    mesh=plsc.VectorSubcoreMesh(core_axis_name="c", subcore_axis_name="s"),
    out_type=(),
    scratch_types=[
        pltpu.VMEM((_LROWS, _CH), jnp.int32),
        pltpu.VMEM((_LROWS, _CH), jnp.int32),
        pltpu.VMEM((128,), jnp.int32),
        pltpu.VMEM((4, _CH, _D), jnp.float32),
        pltpu.SemaphoreType.DMA,
        pltpu.SemaphoreType.DMA,
    ],
    compiler_params=pltpu.CompilerParams(needs_layout_passes=False),
)
def _emit_k(el_hbm, bl_hbm, cnt_hbm, rows_hbm, out_hbm,
            el2, bl2, cnt_v, rowbuf, sem_g, sem_s):
    wid = lax.axis_index("s") * _NC + lax.axis_index("c")
    pltpu.sync_copy(el_hbm.at[wid], el2)
    pltpu.sync_copy(bl_hbm.at[wid], bl2)
    pltpu.sync_copy(cnt_hbm.at[wid], cnt_v)
    zv = jnp.zeros((16,), jnp.int32)
    n = lax.reduce_max(plsc.load_gather(cnt_v, [zv]), (0,))
    nch = lax.shift_right_logical(n + 127, 7)

    gh = [None] * _LROWS
    sh = [None] * _LROWS
    for c in range(_LROWS + 1):
        if c >= 4 and sh[c - 4] is not None:
            @pl.when(c - 4 < nch)
            def _(c=c):
                sh[c - 4].wait()
        if c < _LROWS:
            gh[c] = pltpu.make_async_copy(
                rows_hbm.at[bl2.at[c]], rowbuf.at[c % 4], sem_g)

            @pl.when(c < nch)
            def _(c=c):
                gh[c].start()
        if c >= 1:
            sh[c - 1] = pltpu.make_async_copy(
                rowbuf.at[(c - 1) % 4], out_hbm.at[el2.at[c - 1]], sem_s)

            @pl.when(c - 1 < nch)
            def _(c=c):
                gh[c - 1].wait()
                sh[c - 1].start()
    for c in range(max(_LROWS - 3, 0), _LROWS):
        @pl.when(c < nch)
        def _(c=c):
            sh[c].wait()


def kernel(inputs, context, table, W_ctx, b_ctx, W_delta, b_delta):
    emb = jnp.take(table, inputs, axis=0)
    rows = _dense(context, emb, W_ctx, b_ctx.reshape(1, _D),
                  W_delta, b_delta.reshape(1, _D))
    el, bl, cnt = _winner_k(inputs)
    tbl = jax.new_ref(table)
    _emit_k(el, bl, cnt, rows, tbl)
    new_table = jax.freeze(tbl)
    return rows, new_table


# speedup vs baseline: 1.0069x; 1.0069x over previous
import functools

import jax
import jax.numpy as jnp
from jax import lax
from jax.experimental import pallas as pl
from jax.experimental.pallas import tpu as pltpu
from jax.experimental.pallas import tpu_sc as plsc

_NUM_EMB = 100000
_D = 128
_B = 16384
_BLK = 2048

_NC = 2
_NS = 16
_NW = _NC * _NS
_EPW = _NUM_EMB // _NW
_WTBL = 3136
_LROWS = 28
_CH = 128


def _dense_body(ctx_ref, emb_ref, wc_ref, bc_ref, wd_ref, bd_ref, out_ref):
    ctx = ctx_ref[...]
    emb = emb_ref[...]
    ct = lax.dot_general(ctx, wc_ref[...], (((1,), (1,)), ((), ())),
                         preferred_element_type=jnp.float32) + bc_ref[...]
    ct = 1.0 / (1.0 + jnp.exp(-ct))
    dl = lax.dot_general(emb, wd_ref[...], (((1,), (1,)), ((), ())),
                         preferred_element_type=jnp.float32) + bd_ref[...]
    dl = dl * ct
    dl = 1.0 / (1.0 + jnp.exp(-dl))
    u = dl * emb + (1.0 - dl) * ct
    nrm = jnp.sqrt(jnp.sum(u * u, axis=1, keepdims=True))
    out_ref[...] = u / jnp.maximum(nrm, 1e-12)


_dense = pl.pallas_call(
    _dense_body,
    grid=(_B // _BLK,),
    in_specs=[
        pl.BlockSpec((_BLK, _D), lambda i: (i, 0)),
        pl.BlockSpec((_BLK, _D), lambda i: (i, 0)),
        pl.BlockSpec((_D, _D), lambda i: (0, 0)),
        pl.BlockSpec((1, _D), lambda i: (0, 0)),
        pl.BlockSpec((_D, _D), lambda i: (0, 0)),
        pl.BlockSpec((1, _D), lambda i: (0, 0)),
    ],
    out_specs=pl.BlockSpec((_BLK, _D), lambda i: (i, 0)),
    out_shape=jax.ShapeDtypeStruct((_B, _D), jnp.float32),
)


@functools.partial(
    pl.kernel,
    mesh=plsc.VectorSubcoreMesh(core_axis_name="c", subcore_axis_name="s"),
    out_type=(
        jax.ShapeDtypeStruct((_NW, _LROWS, _CH), jnp.int32),
        jax.ShapeDtypeStruct((_NW, _LROWS, _CH), jnp.int32),
        jax.ShapeDtypeStruct((_NW, 128), jnp.int32),
    ),
    scratch_types=[
        pltpu.VMEM((_B,), jnp.int32),
        pltpu.VMEM((_WTBL,), jnp.int32),
        pltpu.VMEM((_LROWS, _CH), jnp.int32),
        pltpu.VMEM((_LROWS, _CH), jnp.int32),
        pltpu.VMEM((128,), jnp.int32),
    ],
    compiler_params=pltpu.CompilerParams(needs_layout_passes=False),
)
def _winner_k(idx_hbm, el_hbm, bl_hbm, cnt_hbm,
              idx_v, wtbl, el2, bl2, cnt_v):
    wid = lax.axis_index("s") * _NC + lax.axis_index("c")
    lo = wid * _EPW
    iot = lax.iota(jnp.int32, 16)

    pltpu.sync_copy(idx_hbm, idx_v)

    zero = jnp.zeros((16,), jnp.int32)

    def _clear(i, c):
        wtbl[pl.ds(i * 16, 16)] = zero
        return c
    lax.fori_loop(0, _WTBL // 16, _clear, 0)

    def _scan(c4, carry):
        for u in range(4):
            c = c4 * 4 + u
            iv = idx_v[pl.ds(c * 16, 16)]
            _, lastm = plsc.scan_count(iv)
            own = (iv >= lo) & (iv < lo + _EPW) & lastm
            loc = jnp.where(own, iv - lo, _EPW)
            bv = c * 16 + iot + 1
            plsc.store_scatter(wtbl, [loc], bv, mask=own)
        return carry
    lax.fori_loop(0, _B // 64, _scan, 0)

    def _compact(c, n):
        wv = wtbl[pl.ds(c * 16, 16)]
        m = wv > 0
        mi = m.astype(jnp.int32)
        ev = lo + c * 16 + iot
        incl = plsc.cumsum(mi)
        pos = n + incl - 1
        r = lax.shift_right_logical(pos, 7)
        col = pos & 127
        plsc.store_scatter(el2, [r, col], ev, mask=m)
        plsc.store_scatter(bl2, [r, col], wv - 1, mask=m)
        return n + lax.reduce_sum(mi, (0,))
    n = lax.fori_loop(0, _WTBL // 16, _compact, jnp.int32(0))

    last = jnp.maximum(n - 1, 0)
    lr = jnp.full((16,), lax.shift_right_logical(last, 7), jnp.int32)
    lc = jnp.full((16,), last & 127, jnp.int32)
    e0 = plsc.load_gather(el2, [lr, lc])
    b0 = plsc.load_gather(bl2, [lr, lc])
    for k in range(8):
        posv = n + k * 16 + iot
        pr = lax.shift_right_logical(posv, 7)
        pc = posv & 127
        plsc.store_scatter(el2, [pr, pc], e0)
        plsc.store_scatter(bl2, [pr, pc], b0)

    cnt_v[pl.ds(0, 16)] = jnp.full((16,), n, jnp.int32)
    pltpu.sync_copy(el2, el_hbm.at[wid])
    pltpu.sync_copy(bl2, bl_hbm.at[wid])
    pltpu.sync_copy(cnt_v, cnt_hbm.at[wid])


@functools.partial(
    pl.kernel,
    mesh=plsc.VectorSubcoreMesh(core_axis_name="c", subcore_axis_name="s"),
    out_type=(),
    scratch_types=[
        pltpu.VMEM((_LROWS, _CH), jnp.int32),
        pltpu.VMEM((_LROWS, _CH), jnp.int32),
        pltpu.VMEM((128,), jnp.int32),
        pltpu.VMEM((4, _CH, _D), jnp.float32),
        pltpu.SemaphoreType.DMA,
        pltpu.SemaphoreType.DMA,
    ],
    compiler_params=pltpu.CompilerParams(needs_layout_passes=False),
)
def _emit_k(el_hbm, bl_hbm, cnt_hbm, rows_hbm, out_hbm,
            el2, bl2, cnt_v, rowbuf, sem_g, sem_s):
    wid = lax.axis_index("s") * _NC + lax.axis_index("c")
    pltpu.sync_copy(el_hbm.at[wid], el2)
    pltpu.sync_copy(bl_hbm.at[wid], bl2)
    pltpu.sync_copy(cnt_hbm.at[wid], cnt_v)
    zv = jnp.zeros((16,), jnp.int32)
    n = lax.reduce_max(plsc.load_gather(cnt_v, [zv]), (0,))
    nch = lax.shift_right_logical(n + 127, 7)

    gh = [None] * _LROWS
    sh = [None] * _LROWS
    for c in range(_LROWS + 1):
        if c >= 4 and sh[c - 4] is not None:
            @pl.when(c - 4 < nch)
            def _(c=c):
                sh[c - 4].wait()
        if c < _LROWS:
            gh[c] = pltpu.make_async_copy(
                rows_hbm.at[bl2.at[c]], rowbuf.at[c % 4], sem_g)

            @pl.when(c < nch)
            def _(c=c):
                gh[c].start()
        if c >= 1:
            sh[c - 1] = pltpu.make_async_copy(
                rowbuf.at[(c - 1) % 4], out_hbm.at[el2.at[c - 1]], sem_s)

            @pl.when(c - 1 < nch)
            def _(c=c):
                gh[c - 1].wait()
                sh[c - 1].start()
    for c in range(max(_LROWS - 3, 0), _LROWS):
        @pl.when(c < nch)
        def _(c=c):
            sh[c].wait()


def kernel(inputs, context, table, W_ctx, b_ctx, W_delta, b_delta):
    emb = jnp.take(table, inputs, axis=0)
    rows = _dense(context, emb, W_ctx, b_ctx.reshape(1, _D),
                  W_delta, b_delta.reshape(1, _D))
    el, bl, cnt = _winner_k(inputs)
    tbl = jax.new_ref(table)
    _emit_k(el, bl, cnt, rows, tbl)
    new_table = jax.freeze(tbl)
    return rows, new_table
